# baseline (speedup 1.0000x reference)
import functools
import math

import jax
import jax.numpy as jnp
from jax import lax
from jax.experimental import pallas as pl
from jax.experimental.pallas import tpu as pltpu
import jax.experimental.pallas.tpu_sc as plsc

_EPS = 1e-6
_L = 16
_NW = 32
_CHUNK = 2048
_INV_PI = 1.0 / math.pi

_COS_COEF = (1.0, -1.23370054, 2.53669357e-01, -2.08627950e-02,
             9.17858614e-04, -2.38830175e-05)
_SIN_COEF = (1.57079633, -6.45964094e-01, 7.96925939e-02, -4.68163687e-03,
             1.60235188e-04, -3.42523940e-06)


def _poly(g, coef):
    acc = jnp.full_like(g, coef[-1])
    for c in coef[-2::-1]:
        acc = acc * g + c
    return acc


def _rsqrt(x, iters=2):
    i = lax.bitcast_convert_type(x, jnp.int32)
    i = jnp.int32(0x5F3759DF) - lax.shift_right_logical(i, 1)
    y = lax.bitcast_convert_type(i, jnp.float32)
    hx = 0.5 * x
    for _ in range(iters):
        y = y * (1.5 - hx * y * y)
    return y


def _sqrt(x):
    return x * _rsqrt(jnp.maximum(x, 1e-30))


def _sincos_2pi(t):
    u = t * 4.0
    q1 = u >= 1.0
    q2 = u >= 2.0
    q3 = u >= 3.0
    kf = (jnp.where(q1, 1.0, 0.0) + jnp.where(q2, 1.0, 0.0)
          + jnp.where(q3, 1.0, 0.0))
    f = u - kf
    g = f * f
    c0 = _poly(g, _COS_COEF)
    s0 = f * _poly(g, _SIN_COEF)
    swap = (q1 & ~q2) | q3
    cosv = jnp.where(swap, s0, c0)
    sinv = jnp.where(swap, c0, s0)
    cneg = q1 & ~q3
    sneg = q2
    cosv = jnp.where(cneg, -cosv, cosv)
    sinv = jnp.where(sneg, -sinv, sinv)
    return sinv, cosv


def _dot(ax, ay, az, bx, by, bz):
    return ax * bx + ay * by + az * bz


def _norm3(x, y, z):
    s = x * x + y * y + z * z
    inv = _rsqrt(jnp.maximum(s, _EPS * _EPS))
    return x * inv, y * inv, z * inv


def _ray_step(wix, wiy, wiz, nx, ny, nz, wox, woy, woz, ax, u1, u2):
    si = wix * wix + wiy * wiy + wiz * wiz
    inv_i = _rsqrt(jnp.maximum(si, _EPS * _EPS), iters=3)
    wix, wiy, wiz = wix * inv_i, wiy * inv_i, wiz * inv_i
    nx, ny, nz = _norm3(nx, ny, nz)
    wox, woy, woz = _norm3(wox, woy, woz)
    a2 = ax * ax
    b = 1.0 - a2
    cond = jnp.abs(wiz) < 0.999
    cx = jnp.where(cond, -wiy, 0.0)
    cy = jnp.where(cond, wix, -wiz)
    cz = jnp.where(cond, 0.0, wiy)
    wkx, wky, wkz = _norm3(cx, cy, cz)
    wjx = wiy * wkz - wiz * wky
    wjy = wiz * wkx - wix * wkz
    wjz = wix * wky - wiy * wkx
    nk = _dot(nx, ny, nz, wkx, wky, wkz)
    nj = _dot(nx, ny, nz, wjx, wjy, wjz)
    ni = _dot(nx, ny, nz, wix, wiy, wiz)
    nn = _dot(nx, ny, nz, nx, ny, nz)
    Skj = b * nk * nj
    Ski = b * nk * ni
    Sji = b * nj * ni
    Skk = a2 + b * nk * nk
    Sjj = a2 + b * nj * nj
    Sii = a2 + b * ni * ni
    d = a2 + b * nn
    detS = jnp.maximum(a2 * a2 * d, _EPS)
    rd = _rsqrt(detS)
    sqrt_detS = detS * rd
    inv_sqrt_Sii = _rsqrt(jnp.maximum(Sii, _EPS))
    tc = jnp.maximum(Sjj * Sii - Sji * Sji, _EPS)
    inv_tmp = _rsqrt(tc)
    tmp = tc * inv_tmp
    Mkx = sqrt_detS * inv_tmp
    Mjx = -inv_sqrt_Sii * (Ski * Sji - Skj * Sii) * inv_tmp
    Mjy = inv_sqrt_Sii * tmp
    Mix = inv_sqrt_Sii * Ski
    Miy = inv_sqrt_Sii * Sji
    Miz = inv_sqrt_Sii * Sii
    r = _sqrt(u1)
    sinp, cosp = _sincos_2pi(u2)
    uu = r * cosp
    vv = r * sinp
    ww = _sqrt(jnp.maximum(1.0 - uu * uu - vv * vv, 0.0))
    x = uu * Mkx + vv * Mjx + ww * Mix
    y = vv * Mjy + ww * Miy
    z = ww * Miz
    x, y, z = _norm3(x, y, z)
    wmx = x * wkx + y * wjx + z * wix
    wmy = x * wky + y * wjy + z * wiy
    wmz = x * wkz + y * wjz + z * wiz
    diff = jnp.maximum(_dot(wox, woy, woz, wmx, wmy, wmz), 0.0) * _INV_PI
    vx = wix + wox
    vy = wiy + woy
    vz = wiz + woz
    vv_ = _dot(vx, vy, vz, vx, vy, vz)
    inv_h = _rsqrt(jnp.maximum(vv_, _EPS * _EPS))
    hh = vv_ * inv_h * inv_h
    nh = _dot(nx, ny, nz, vx, vy, vz) * inv_h
    q = jnp.maximum((hh * d - b * nh * nh) / (a2 * d), _EPS)
    spec = (0.25 * _INV_PI) * rd * inv_sqrt_Sii / (q * q)
    return diff, spec


def _sc_body(wi_hbm, n_hbm, wo_hbm, ax_hbm, samp_hbm, diff_hbm, spec_hbm,
             wi_v, n_v, wo_v, ax_v, samp_v, diff_v, spec_v, *, n_rays):
    wid = lax.axis_index("s") * 2 + lax.axis_index("c")
    per_w = n_rays // _NW
    n_chunks = per_w // _CHUNK
    iota = lax.broadcasted_iota(jnp.int32, (_L,), 0)

    def chunk_body(c, _):
        base = wid * per_w + c * _CHUNK
        pltpu.sync_copy(wi_hbm.at[pl.ds(base * 3, _CHUNK * 3)], wi_v)
        pltpu.sync_copy(n_hbm.at[pl.ds(base * 3, _CHUNK * 3)], n_v)
        pltpu.sync_copy(wo_hbm.at[pl.ds(base * 3, _CHUNK * 3)], wo_v)
        pltpu.sync_copy(ax_hbm.at[pl.ds(base, _CHUNK)], ax_v)
        pltpu.sync_copy(samp_hbm.at[pl.ds(base * 2, _CHUNK * 2)], samp_v)

        @plsc.parallel_loop(0, _CHUNK // _L, step=1, unroll=4)
        def step(i):
            rows = i * _L + iota
            r3 = rows * 3
            r2 = rows * 2
            wix = plsc.load_gather(wi_v, [r3])
            wiy = plsc.load_gather(wi_v, [r3 + 1])
            wiz = plsc.load_gather(wi_v, [r3 + 2])
            nx = plsc.load_gather(n_v, [r3])
            ny = plsc.load_gather(n_v, [r3 + 1])
            nz = plsc.load_gather(n_v, [r3 + 2])
            wox = plsc.load_gather(wo_v, [r3])
            woy = plsc.load_gather(wo_v, [r3 + 1])
            woz = plsc.load_gather(wo_v, [r3 + 2])
            u1 = plsc.load_gather(samp_v, [r2])
            u2 = plsc.load_gather(samp_v, [r2 + 1])
            ax = ax_v[pl.ds(i * _L, _L)]
            diff, spec = _ray_step(wix, wiy, wiz, nx, ny, nz,
                                   wox, woy, woz, ax, u1, u2)
            diff_v[pl.ds(i * _L, _L)] = diff
            spec_v[pl.ds(i * _L, _L)] = spec

        pltpu.sync_copy(diff_v, diff_hbm.at[pl.ds(base, _CHUNK)])
        pltpu.sync_copy(spec_v, spec_hbm.at[pl.ds(base, _CHUNK)])
        return ()

    lax.fori_loop(0, n_chunks, chunk_body, ())


@functools.partial(jax.jit, static_argnames=())
def kernel(wi, n, wo, alpha_x, alpha_y, sample):
    del alpha_y
    n_rays = wi.shape[0]
    mesh = plsc.VectorSubcoreMesh(core_axis_name="c", subcore_axis_name="s")
    f32 = jnp.float32
    run = pl.kernel(
        functools.partial(_sc_body, n_rays=n_rays),
        out_type=[jax.ShapeDtypeStruct((n_rays,), f32),
                  jax.ShapeDtypeStruct((n_rays,), f32)],
        mesh=mesh,
        scratch_types=[
            pltpu.VMEM((_CHUNK * 3,), f32),
            pltpu.VMEM((_CHUNK * 3,), f32),
            pltpu.VMEM((_CHUNK * 3,), f32),
            pltpu.VMEM((_CHUNK,), f32),
            pltpu.VMEM((_CHUNK * 2,), f32),
            pltpu.VMEM((_CHUNK,), f32),
            pltpu.VMEM((_CHUNK,), f32),
        ],
        compiler_params=pltpu.CompilerParams(needs_layout_passes=False),
    )
    diff, spec = run(wi.reshape(-1), n.reshape(-1), wo.reshape(-1),
                     alpha_x, sample.reshape(-1))
    return diff[:, None], spec[:, None]


# speedup vs baseline: 13.2456x; 13.2456x over previous
import functools
import math

import jax
import jax.numpy as jnp
from jax import lax
from jax.experimental import pallas as pl
from jax.experimental.pallas import tpu as pltpu
import jax.experimental.pallas.tpu_sc as plsc

_EPS = 1e-6
_L = 16
_NW = 32
_CHUNK = 2048
_INV_PI = 1.0 / math.pi

_COS_COEF = (1.0, -1.23370054, 2.53669357e-01, -2.08627950e-02,
             9.17858614e-04, -2.38830175e-05)
_SIN_COEF = (1.57079633, -6.45964094e-01, 7.96925939e-02, -4.68163687e-03,
             1.60235188e-04, -3.42523940e-06)


def _poly(g, coef):
    acc = jnp.full_like(g, coef[-1])
    for c in coef[-2::-1]:
        acc = acc * g + c
    return acc


def _rsqrt(x, iters=2):
    i = lax.bitcast_convert_type(x, jnp.int32)
    i = jnp.int32(0x5F3759DF) - lax.shift_right_logical(i, 1)
    y = lax.bitcast_convert_type(i, jnp.float32)
    hx = 0.5 * x
    for _ in range(iters):
        y = y * (1.5 - hx * y * y)
    return y


def _sqrt(x):
    return x * _rsqrt(jnp.maximum(x, 1e-30))


def _sincos_2pi(t):
    u = t * 4.0
    q1 = u >= 1.0
    q2 = u >= 2.0
    q3 = u >= 3.0
    kf = (jnp.where(q1, 1.0, 0.0) + jnp.where(q2, 1.0, 0.0)
          + jnp.where(q3, 1.0, 0.0))
    f = u - kf
    g = f * f
    c0 = _poly(g, _COS_COEF)
    s0 = f * _poly(g, _SIN_COEF)
    swap = (q1 & ~q2) | q3
    cosv = jnp.where(swap, s0, c0)
    sinv = jnp.where(swap, c0, s0)
    cneg = q1 & ~q3
    sneg = q2
    cosv = jnp.where(cneg, -cosv, cosv)
    sinv = jnp.where(sneg, -sinv, sinv)
    return sinv, cosv


def _dot(ax, ay, az, bx, by, bz):
    return ax * bx + ay * by + az * bz


def _norm3(x, y, z):
    s = x * x + y * y + z * z
    inv = _rsqrt(jnp.maximum(s, _EPS * _EPS))
    return x * inv, y * inv, z * inv


def _ray_step(wix, wiy, wiz, nx, ny, nz, wox, woy, woz, ax, u1, u2):
    si = wix * wix + wiy * wiy + wiz * wiz
    inv_i = _rsqrt(jnp.maximum(si, _EPS * _EPS), iters=3)
    wix, wiy, wiz = wix * inv_i, wiy * inv_i, wiz * inv_i
    nx, ny, nz = _norm3(nx, ny, nz)
    wox, woy, woz = _norm3(wox, woy, woz)
    a2 = ax * ax
    b = 1.0 - a2
    cond = jnp.abs(wiz) < 0.999
    cx = jnp.where(cond, -wiy, 0.0)
    cy = jnp.where(cond, wix, -wiz)
    cz = jnp.where(cond, 0.0, wiy)
    wkx, wky, wkz = _norm3(cx, cy, cz)
    wjx = wiy * wkz - wiz * wky
    wjy = wiz * wkx - wix * wkz
    wjz = wix * wky - wiy * wkx
    nk = _dot(nx, ny, nz, wkx, wky, wkz)
    nj = _dot(nx, ny, nz, wjx, wjy, wjz)
    ni = _dot(nx, ny, nz, wix, wiy, wiz)
    nn = _dot(nx, ny, nz, nx, ny, nz)
    Skj = b * nk * nj
    Ski = b * nk * ni
    Sji = b * nj * ni
    Skk = a2 + b * nk * nk
    Sjj = a2 + b * nj * nj
    Sii = a2 + b * ni * ni
    d = a2 + b * nn
    detS = jnp.maximum(a2 * a2 * d, _EPS)
    rd = _rsqrt(detS)
    sqrt_detS = detS * rd
    inv_sqrt_Sii = _rsqrt(jnp.maximum(Sii, _EPS))
    tc = jnp.maximum(Sjj * Sii - Sji * Sji, _EPS)
    inv_tmp = _rsqrt(tc)
    tmp = tc * inv_tmp
    Mkx = sqrt_detS * inv_tmp
    Mjx = -inv_sqrt_Sii * (Ski * Sji - Skj * Sii) * inv_tmp
    Mjy = inv_sqrt_Sii * tmp
    Mix = inv_sqrt_Sii * Ski
    Miy = inv_sqrt_Sii * Sji
    Miz = inv_sqrt_Sii * Sii
    r = _sqrt(u1)
    sinp, cosp = _sincos_2pi(u2)
    uu = r * cosp
    vv = r * sinp
    ww = _sqrt(jnp.maximum(1.0 - uu * uu - vv * vv, 0.0))
    x = uu * Mkx + vv * Mjx + ww * Mix
    y = vv * Mjy + ww * Miy
    z = ww * Miz
    x, y, z = _norm3(x, y, z)
    wmx = x * wkx + y * wjx + z * wix
    wmy = x * wky + y * wjy + z * wiy
    wmz = x * wkz + y * wjz + z * wiz
    diff = jnp.maximum(_dot(wox, woy, woz, wmx, wmy, wmz), 0.0) * _INV_PI
    vx = wix + wox
    vy = wiy + woy
    vz = wiz + woz
    vv_ = _dot(vx, vy, vz, vx, vy, vz)
    inv_h = _rsqrt(jnp.maximum(vv_, _EPS * _EPS))
    hh = vv_ * inv_h * inv_h
    nh = _dot(nx, ny, nz, vx, vy, vz) * inv_h
    q = jnp.maximum((hh * d - b * nh * nh) / (a2 * d), _EPS)
    spec = (0.25 * _INV_PI) * rd * inv_sqrt_Sii / (q * q)
    return diff, spec


def _sc_body(*refs, n_rays):
    in_hbm = refs[:12]
    diff_hbm, spec_hbm = refs[12], refs[13]
    in_v = refs[14:26]
    diff_v, spec_v = refs[26], refs[27]
    wid = lax.axis_index("s") * 2 + lax.axis_index("c")
    per_w = n_rays // _NW
    n_chunks = per_w // _CHUNK

    def chunk_body(c, _):
        base = wid * per_w + c * _CHUNK
        for h, v in zip(in_hbm, in_v):
            pltpu.sync_copy(h.at[pl.ds(base, _CHUNK)], v)

        @plsc.parallel_loop(0, _CHUNK // _L, step=1, unroll=4)
        def step(i):
            o = i * _L
            vals = [v[pl.ds(o, _L)] for v in in_v]
            diff, spec = _ray_step(*vals)
            diff_v[pl.ds(o, _L)] = diff
            spec_v[pl.ds(o, _L)] = spec

        pltpu.sync_copy(diff_v, diff_hbm.at[pl.ds(base, _CHUNK)])
        pltpu.sync_copy(spec_v, spec_hbm.at[pl.ds(base, _CHUNK)])
        return ()

    lax.fori_loop(0, n_chunks, chunk_body, ())


@functools.partial(jax.jit, static_argnames=())
def kernel(wi, n, wo, alpha_x, alpha_y, sample):
    del alpha_y
    n_rays = wi.shape[0]
    mesh = plsc.VectorSubcoreMesh(core_axis_name="c", subcore_axis_name="s")
    f32 = jnp.float32
    run = pl.kernel(
        functools.partial(_sc_body, n_rays=n_rays),
        out_type=[jax.ShapeDtypeStruct((n_rays,), f32),
                  jax.ShapeDtypeStruct((n_rays,), f32)],
        mesh=mesh,
        scratch_types=(
            [pltpu.VMEM((_CHUNK,), f32)] * 14
        ),
        compiler_params=pltpu.CompilerParams(needs_layout_passes=False),
    )
    diff, spec = run(wi[:, 0], wi[:, 1], wi[:, 2],
                     n[:, 0], n[:, 1], n[:, 2],
                     wo[:, 0], wo[:, 1], wo[:, 2],
                     alpha_x, sample[:, 0], sample[:, 1])
    return diff[:, None], spec[:, None]
